# Initial kernel scaffold; baseline (speedup 1.0000x reference)
import jax
import jax.numpy as jnp
from jax.experimental import pallas as pl


def kernel(efeat, nfeat, edge_index, edge_W1, edge_b1, edge_W2, edge_b2, edge_g, edge_beta, node_W1, node_b1, node_W2, node_b2, node_g, node_beta):
    raise NotImplementedError("write your pallas kernel here")


# speedup vs baseline: 2.0322x; 2.0322x over previous
import functools

import jax
import jax.numpy as jnp
from jax import lax
from jax.experimental import pallas as pl
from jax.experimental.pallas import tpu as pltpu
from jax.experimental.pallas import tpu_sc as plsc

N = 10000
E = 320000
H = 128
L = 16

NC = 2
NS = 16
NW = NC * NS
C = 128
K = 80
EPAD = NW * K * C
RPW = K * C
NHALF = 5120
NACC = 5248
NPT = NACC // NS
K2 = EPAD // NS // C

EB = 2560
_EPS = 1e-5


def _edge_body(ef_ref, gs_ref, gd_ref, w1_ref, w2_ref, b1_ref, b2_ref,
               gam_ref, bet_ref, out_ref):
    ef = ef_ref[...]
    pre = (jnp.dot(ef, w1_ref[...], preferred_element_type=jnp.float32)
           + gs_ref[...] + gd_ref[...] + b1_ref[...])
    h = pre * jax.nn.sigmoid(pre)
    y = jnp.dot(h, w2_ref[...], preferred_element_type=jnp.float32) + b2_ref[...]
    m = jnp.mean(y, axis=-1, keepdims=True)
    d = y - m
    v = jnp.mean(d * d, axis=-1, keepdims=True)
    ln = d * lax.rsqrt(v + _EPS) * gam_ref[...] + bet_ref[...]
    rows = pl.program_id(0) * EB + lax.broadcasted_iota(jnp.int32, (EB, 1), 0)
    out_ref[...] = ef + jnp.where(rows < E, ln, 0.0)


@functools.lru_cache(maxsize=None)
def _edge_call():
    row = pl.BlockSpec((EB, H), lambda i: (i, 0))
    mat = pl.BlockSpec((H, H), lambda i: (0, 0))
    vec = pl.BlockSpec((1, H), lambda i: (0, 0))
    return pl.pallas_call(
        _edge_body,
        grid=(EPAD // EB,),
        in_specs=[row, row, row, mat, mat, vec, vec, vec, vec],
        out_specs=row,
        out_shape=jax.ShapeDtypeStruct((EPAD, H), jnp.float32),
    )


def _node_body(agg_ref, nf_ref, wa_ref, wn_ref, w2_ref, b1_ref, b2_ref,
               gam_ref, bet_ref, out_ref):
    agg = jnp.concatenate([agg_ref[0, :NHALF], agg_ref[1, :N - NHALF]], axis=0)
    nf = nf_ref[...]
    pre = (jnp.dot(agg, wa_ref[...], preferred_element_type=jnp.float32)
           + jnp.dot(nf, wn_ref[...], preferred_element_type=jnp.float32)
           + b1_ref[...])
    h = pre * jax.nn.sigmoid(pre)
    y = jnp.dot(h, w2_ref[...], preferred_element_type=jnp.float32) + b2_ref[...]
    m = jnp.mean(y, axis=-1, keepdims=True)
    d = y - m
    v = jnp.mean(d * d, axis=-1, keepdims=True)
    out_ref[...] = nf + d * lax.rsqrt(v + _EPS) * gam_ref[...] + bet_ref[...]


@functools.lru_cache(maxsize=None)
def _node_call():
    return pl.pallas_call(
        _node_body,
        out_shape=jax.ShapeDtypeStruct((N, H), jnp.float32),
    )


def _proj_body(nf_ref, ws_ref, wd_ref, ps_ref, pd_ref):
    nf = nf_ref[...]
    ps_ref[...] = jnp.dot(nf, ws_ref[...], preferred_element_type=jnp.float32)
    pd_ref[...] = jnp.dot(nf, wd_ref[...], preferred_element_type=jnp.float32)


@functools.lru_cache(maxsize=None)
def _proj_call():
    return pl.pallas_call(
        _proj_body,
        out_shape=(jax.ShapeDtypeStruct((N, H), jnp.float32),
                   jax.ShapeDtypeStruct((N, H), jnp.float32)),
    )


NBG = 3
NBS = 4


@functools.lru_cache(maxsize=None)
def _gather_call():
    mesh = plsc.VectorSubcoreMesh(core_axis_name="c", subcore_axis_name="s",
                                  num_cores=NC, num_subcores=NS)

    @functools.partial(
        pl.kernel,
        out_type=(jax.ShapeDtypeStruct((EPAD, H), jnp.float32),
                  jax.ShapeDtypeStruct((EPAD, H), jnp.float32)),
        mesh=mesh,
        scratch_types=(
            [pltpu.VMEM((K, C), jnp.int32),
             pltpu.VMEM((K, C), jnp.int32),
             pltpu.VMEM((2, NBG, C, H), jnp.float32)]
            + [pltpu.SemaphoreType.DMA] * (4 * NBG)
        ),
    )
    def gather(ps_hbm, pd_hbm, src_hbm, dst_hbm, gs_hbm, gd_hbm,
               src_v, dst_v, buf_v, *sems):
        sem_in = (sems[:NBG], sems[NBG:2 * NBG])
        sem_out = (sems[2 * NBG:3 * NBG], sems[3 * NBG:])
        w = lax.axis_index("s") * NC + lax.axis_index("c")
        rz = pl.multiple_of(jnp.minimum(w, 0), 8)
        pltpu.sync_copy(src_hbm.at[w, pl.ds(rz, K)], src_v)
        pltpu.sync_copy(dst_hbm.at[w, pl.ds(rz, K)], dst_v)
        base = w * RPW
        tbls = (ps_hbm, pd_hbm)
        idxs = (src_v, dst_v)
        outs = (gs_hbm, gd_hbm)

        def fire_in(t, b, j):
            pltpu.async_copy(tbls[t].at[idxs[t].at[j]], buf_v.at[t, b],
                             sem_in[t][b])

        def wait_in(t, b, j):
            pltpu.make_async_copy(tbls[t].at[idxs[t].at[j]], buf_v.at[t, b],
                                  sem_in[t][b]).wait()

        def fire_out(t, b, j):
            pltpu.async_copy(buf_v.at[t, b],
                             outs[t].at[pl.ds(base + j * C, C)], sem_out[t][b])

        def wait_out(t, b, j):
            pltpu.make_async_copy(buf_v.at[t, b],
                                  outs[t].at[pl.ds(base + j * C, C)],
                                  sem_out[t][b]).wait()

        for t in (0, 1):
            for b in range(NBG):
                fire_in(t, b, rz + b)

        GFULL = (K - 2 * NBG) // NBG + 1

        def grp(g, carry):
            j0 = g * NBG
            for t in (0, 1):
                for b in range(NBG):
                    wait_in(t, b, j0 + b)
                    fire_out(t, b, j0 + b)
                for b in range(NBG):
                    wait_out(t, b, j0 + b)
                    fire_in(t, b, j0 + NBG + b)
            return carry

        lax.fori_loop(0, GFULL, grp, 0)
        jstart = GFULL * NBG
        for t in (0, 1):
            last = []
            for i, jj in enumerate(range(jstart, K)):
                b = i % NBG
                j = rz + jj
                wait_in(t, b, j)
                fire_out(t, b, j)
                if jj + NBG < K:
                    wait_out(t, b, j)
                    fire_in(t, b, j + NBG)
                else:
                    last.append((b, j))
            for b, j in last:
                wait_out(t, b, j)

    return gather


@functools.lru_cache(maxsize=None)
def _scatter_call():
    mesh = plsc.VectorSubcoreMesh(core_axis_name="c", subcore_axis_name="s",
                                  num_cores=NC, num_subcores=NS)

    @functools.partial(
        pl.kernel,
        out_type=jax.ShapeDtypeStruct((NC, NACC, H), jnp.float32),
        mesh=mesh,
        scratch_types=(
            [pltpu.VMEM((K2, C), jnp.int32),
             pltpu.VMEM((NBS, C), jnp.int32),
             pltpu.VMEM((NBS, C, H), jnp.float32),
             pltpu.VMEM_SHARED((NACC, H), jnp.float32)]
            + [pltpu.SemaphoreType.DMA] * (2 * NBS)
        ),
    )
    def scatter(e_hbm, dst_hbm, z_hbm, out_hbm,
                dst_v, loc_v, rows_v, acc_sh, *sems):
        sem_r = sems[:NBS]
        sem_a = sems[NBS:]
        c = lax.axis_index("c")
        s = lax.axis_index("s")
        rz = pl.multiple_of(jnp.minimum(s, 0), 8)

        def inner(acc_sh):
            pltpu.sync_copy(z_hbm.at[pl.ds(rz, C)], rows_v.at[0])
            off = 0
            while off < NPT:
                sz = min(C, NPT - off)
                pltpu.sync_copy(rows_v.at[0, pl.ds(0, sz)],
                                acc_sh.at[pl.ds(s * NPT + off, sz)])
                off += sz
            pltpu.sync_copy(dst_hbm.at[s, pl.ds(rz, K2)], dst_v)
            lo = c * NHALF
            plsc.subcore_barrier()
            base = s * K2 * C

            def fire_rows(b, j):
                pltpu.async_copy(e_hbm.at[pl.ds(base + j * C, C)],
                                 rows_v.at[b], sem_r[b])

            def wait_rows(b, j):
                pltpu.make_async_copy(e_hbm.at[pl.ds(base + j * C, C)],
                                      rows_v.at[b], sem_r[b]).wait()

            def localize(b, j):
                for v in range(C // 16):
                    d = dst_v[j, pl.ds(v * 16, 16)]
                    local = d - lo
                    ok = (local >= 0) & (local < NHALF)
                    loc_v[b, pl.ds(v * 16, 16)] = jnp.where(ok, local, NHALF)

            def fire_add(b):
                pltpu.async_copy(rows_v.at[b], acc_sh.at[loc_v.at[b]],
                                 sem_a[b], add=True)

            def wait_add(b):
                pltpu.make_async_copy(rows_v.at[b], acc_sh.at[loc_v.at[b]],
                                      sem_a[b]).wait()

            for b in range(NBS):
                fire_rows(b, rz + b)

            def grp(g, carry):
                j0 = g * NBS
                for b in range(NBS):
                    wait_rows(b, j0 + b)
                    localize(b, j0 + b)
                    fire_add(b)
                for b in range(NBS):
                    wait_add(b)
                    fire_rows(b, j0 + NBS + b)
                return carry

            lax.fori_loop(0, K2 // NBS - 1, grp, 0)
            j0 = rz + (K2 - NBS)
            for b in range(NBS):
                wait_rows(b, j0 + b)
                localize(b, j0 + b)
                fire_add(b)
            for b in range(NBS):
                wait_add(b)
            plsc.subcore_barrier()
            off = 0
            while off < NPT:
                sz = min(C, NPT - off)
                pltpu.sync_copy(acc_sh.at[pl.ds(s * NPT + off, sz)],
                                rows_v.at[0, pl.ds(0, sz)])
                pltpu.sync_copy(rows_v.at[0, pl.ds(0, sz)],
                                out_hbm.at[c, pl.ds(s * NPT + off, sz)])
                off += sz

        inner(acc_sh)

    return scatter


def kernel(efeat, nfeat, edge_index, edge_W1, edge_b1, edge_W2, edge_b2,
           edge_g, edge_beta, node_W1, node_b1, node_W2, node_b2,
           node_g, node_beta):
    pad = EPAD - E
    ef = jnp.pad(efeat, ((0, pad), (0, 0)))
    srcp = jnp.pad(edge_index[0], (0, pad)).reshape(NW, K, C)
    dstp = jnp.pad(edge_index[1], (0, pad)).reshape(NW, K, C)
    dstp16 = dstp.reshape(NS, K2, C)
    zrows = jnp.zeros((C, H), jnp.float32)
    nf = nfeat

    edge = _edge_call()
    node = _node_call()
    proj = _proj_call()
    gather = _gather_call()
    scatter = _scatter_call()

    for i in range(L):
        w1e = edge_W1[i, :H]
        w1s = edge_W1[i, H:2 * H]
        w1d = edge_W1[i, 2 * H:]
        ps, pd = proj(nf, w1s, w1d)
        gs, gd = gather(ps, pd, srcp, dstp)
        ef = edge(ef, gs, gd, w1e, edge_W2[i],
                  edge_b1[i].reshape(1, H), edge_b2[i].reshape(1, H),
                  edge_g[i].reshape(1, H), edge_beta[i].reshape(1, H))
        aggp = scatter(ef, dstp16, zrows)
        nf = node(aggp, nf, node_W1[i, :H], node_W1[i, H:], node_W2[i],
                  node_b1[i].reshape(1, H), node_b2[i].reshape(1, H),
                  node_g[i].reshape(1, H), node_beta[i].reshape(1, H))

    return ef[:E], nf
